# baseline (speedup 1.0000x reference)
import functools

import jax
import jax.numpy as jnp
from jax import lax
from jax.experimental import pallas as pl
from jax.experimental.pallas import tpu as pltpu
from jax.experimental.pallas import tpu_sc as plsc

_K = 10.0
_EPS = 1e-9

_NC = 2
_NS = 16
_NW = _NC * _NS
_G = 1024


def _sc_piou(nsc, bpw):
    mesh = plsc.VectorSubcoreMesh(core_axis_name="c", subcore_axis_name="s")

    @functools.partial(
        pl.kernel,
        mesh=mesh,
        out_type=jax.ShapeDtypeStruct((nsc,), jnp.float32),
        scratch_types=[
            pltpu.VMEM((12, bpw), jnp.float32),
            pltpu.VMEM((_G,), jnp.float32),
            pltpu.VMEM((_G,), jnp.float32),
            pltpu.VMEM((bpw,), jnp.float32),
        ],
    )
    def body(params_hbm, gx_hbm, gy_hbm, out_hbm, pv, gxv, gyv, ov):
        wid = lax.axis_index("s") * _NC + lax.axis_index("c")
        base = wid * bpw
        pltpu.sync_copy(params_hbm.at[:, pl.ds(base, bpw)], pv)
        pltpu.sync_copy(gx_hbm, gxv)
        pltpu.sync_copy(gy_hbm, gyv)

        def box_group(j, _):
            o = j * 16
            cxp = pv[0, pl.ds(o, 16)]
            cyp = pv[1, pl.ds(o, 16)]
            ctp = pv[2, pl.ds(o, 16)]
            stp = pv[3, pl.ds(o, 16)]
            wp = pv[4, pl.ds(o, 16)]
            hp = pv[5, pl.ds(o, 16)]
            cxt = pv[6, pl.ds(o, 16)]
            cyt = pv[7, pl.ds(o, 16)]
            ctt = pv[8, pl.ds(o, 16)]
            stt = pv[9, pl.ds(o, 16)]
            wt = pv[10, pl.ds(o, 16)]
            ht = pv[11, pl.ds(o, 16)]

            def gchunk(c, carry):
                acc_i, acc_u = carry
                gxc = gxv[pl.ds(c * 16, 16)]
                gyc = gyv[pl.ds(c * 16, 16)]
                for u in range(16):
                    gxs = gxc[u]
                    gys = gyc[u]
                    dxp = gxs - cxp
                    dyp = gys - cyp
                    ap = jnp.minimum(jnp.abs(dxp * ctp + dyp * stp) - wp, 20.0)
                    bp = jnp.minimum(jnp.abs(dyp * ctp - dxp * stp) - hp, 20.0)
                    dxt = gxs - cxt
                    dyt = gys - cyt
                    at = jnp.minimum(jnp.abs(dxt * ctt + dyt * stt) - wt, 20.0)
                    bt = jnp.minimum(jnp.abs(dyt * ctt - dxt * stt) - ht, 20.0)
                    dp = (1.0 + jnp.exp(ap)) * (1.0 + jnp.exp(bp))
                    dt = (1.0 + jnp.exp(at)) * (1.0 + jnp.exp(bt))
                    r = 1.0 / (dp * dt)
                    acc_i = acc_i + r
                    acc_u = acc_u + (dp + dt - 1.0) * r
                return acc_i, acc_u

            z = jnp.zeros((16,), jnp.float32)
            acc_i, acc_u = lax.fori_loop(0, _G // 16, gchunk, (z, z))
            ov[pl.ds(o, 16)] = acc_i / (acc_u + _EPS)
            return 0

        lax.fori_loop(0, bpw // 16, box_group, 0)
        pltpu.sync_copy(ov, out_hbm.at[pl.ds(base, bpw)])

    return body


def _derived_params(loc):
    cx = loc[:, 0]
    cy = loc[:, 1]
    kw2 = (0.5 * _K) * loc[:, 2]
    kh2 = (0.5 * _K) * loc[:, 3]
    kct = _K * jnp.cos(loc[:, 4])
    kst = _K * jnp.sin(loc[:, 4])
    return [cx, cy, kct, kst, kw2, kh2]


def kernel(loc_p, loc_t, grid):
    n = loc_p.shape[0]
    unit = 16 * _NW
    nsc = ((n + unit - 1) // unit) * unit
    pad = nsc - n
    lp = jnp.pad(loc_p, ((0, pad), (0, 0)))
    lt = jnp.pad(loc_t, ((0, pad), (0, 0)))
    params = jnp.stack(_derived_params(lp) + _derived_params(lt), axis=0)
    gx = grid[:, 0]
    gy = grid[:, 1]
    out = _sc_piou(nsc, nsc // _NW)(params, gx, gy)
    return out[:n]


# speedup vs baseline: 4.5830x; 4.5830x over previous
import functools

import jax
import jax.numpy as jnp
from jax import lax
from jax.experimental import pallas as pl
from jax.experimental.pallas import tpu as pltpu
from jax.experimental.pallas import tpu_sc as plsc

_K = 10.0
_EPS = 1e-9

_NC = 2
_NS = 16
_NW = _NC * _NS
_G = 1024


def _sc_piou(nsc, bpw):
    mesh = plsc.VectorSubcoreMesh(core_axis_name="c", subcore_axis_name="s")

    @functools.partial(
        pl.kernel,
        mesh=mesh,
        out_type=jax.ShapeDtypeStruct((nsc,), jnp.float32),
        scratch_types=[
            pltpu.VMEM((12, bpw), jnp.float32),
            pltpu.VMEM((_G,), jnp.float32),
            pltpu.VMEM((_G,), jnp.float32),
            pltpu.VMEM((bpw,), jnp.float32),
        ],
    )
    def body(params_hbm, gx_hbm, gy_hbm, out_hbm, pv, gxv, gyv, ov):
        wid = lax.axis_index("s") * _NC + lax.axis_index("c")
        base = wid * bpw
        for j in range(12):
            pltpu.sync_copy(params_hbm.at[pl.ds(j * nsc + base, bpw)],
                            pv.at[j])
        pltpu.sync_copy(gx_hbm, gxv)
        pltpu.sync_copy(gy_hbm, gyv)

        def box_group(j, _):
            o = j * 16
            cxp = pv[0, pl.ds(o, 16)]
            cyp = pv[1, pl.ds(o, 16)]
            ctp = pv[2, pl.ds(o, 16)]
            stp = pv[3, pl.ds(o, 16)]
            wp = pv[4, pl.ds(o, 16)]
            hp = pv[5, pl.ds(o, 16)]
            cxt = pv[6, pl.ds(o, 16)]
            cyt = pv[7, pl.ds(o, 16)]
            ctt = pv[8, pl.ds(o, 16)]
            stt = pv[9, pl.ds(o, 16)]
            wt = pv[10, pl.ds(o, 16)]
            ht = pv[11, pl.ds(o, 16)]

            def gchunk(c, carry):
                accs = list(carry)
                gxc = gxv[pl.ds(c * 16, 16)]
                gyc = gyv[pl.ds(c * 16, 16)]
                for u in range(16):
                    gxs = gxc[u]
                    gys = gyc[u]
                    dxp = gxs - cxp
                    dyp = gys - cyp
                    ap = jnp.minimum(jnp.abs(dxp * ctp + dyp * stp) - wp, 20.0)
                    bp = jnp.minimum(jnp.abs(dyp * ctp - dxp * stp) - hp, 20.0)
                    dxt = gxs - cxt
                    dyt = gys - cyt
                    at = jnp.minimum(jnp.abs(dxt * ctt + dyt * stt) - wt, 20.0)
                    bt = jnp.minimum(jnp.abs(dyt * ctt - dxt * stt) - ht, 20.0)
                    dp = (1.0 + jnp.exp(ap)) * (1.0 + jnp.exp(bp))
                    dt = (1.0 + jnp.exp(at)) * (1.0 + jnp.exp(bt))
                    r = 1.0 / (dp * dt)
                    k = u % 4
                    accs[k] = accs[k] + r
                    accs[4 + k] = accs[4 + k] + (dp + dt - 1.0) * r
                return tuple(accs)

            z = jnp.zeros((16,), jnp.float32)
            accs = lax.fori_loop(0, _G // 16, gchunk, (z,) * 8)
            acc_i = (accs[0] + accs[1]) + (accs[2] + accs[3])
            acc_u = (accs[4] + accs[5]) + (accs[6] + accs[7])
            ov[pl.ds(o, 16)] = acc_i / (acc_u + _EPS)
            return 0

        lax.fori_loop(0, bpw // 16, box_group, 0)
        pltpu.sync_copy(ov, out_hbm.at[pl.ds(base, bpw)])

    return body


def _derived_params(loc):
    cx = loc[:, 0]
    cy = loc[:, 1]
    kw2 = (0.5 * _K) * loc[:, 2]
    kh2 = (0.5 * _K) * loc[:, 3]
    kct = _K * jnp.cos(loc[:, 4])
    kst = _K * jnp.sin(loc[:, 4])
    return [cx, cy, kct, kst, kw2, kh2]


def _tc_body(locp_ref, loct_ref, grid_ref, out_ref):
    gx = grid_ref[:, 0:1]
    gy = grid_ref[:, 1:2]
    kl2 = _K * 1.4426950408889634

    def denom(loc):
        cx = loc[0:1, :]
        cy = loc[1:2, :]
        th = loc[4:5, :]
        kct = kl2 * jnp.cos(th)
        kst = kl2 * jnp.sin(th)
        kw2 = (0.5 * kl2) * loc[2:3, :]
        kh2 = (0.5 * kl2) * loc[3:4, :]
        dx = gx - cx
        dy = gy - cy
        a = jnp.abs(dx * kct + dy * kst) - kw2
        b = jnp.abs(dy * kct - dx * kst) - kh2
        ea = jnp.exp2(jnp.minimum(a, 29.0))
        eb = jnp.exp2(jnp.minimum(b, 29.0))
        return (1.0 + ea) * (1.0 + eb)

    dp = denom(locp_ref[...])
    dt = denom(loct_ref[...])
    r = 1.0 / (dp * dt)
    inter = jnp.sum(r, axis=0)
    union = jnp.sum((dp + dt - 1.0) * r, axis=0)
    out_ref[...] = (inter / (union + _EPS))[None, None, :]


def _tc_piou(lp, lt, grid, blk):
    ntc = lp.shape[1]
    g = grid.shape[0]
    nblk = ntc // blk
    out = pl.pallas_call(
        _tc_body,
        grid=(nblk,),
        in_specs=[
            pl.BlockSpec((5, blk), lambda i: (0, i)),
            pl.BlockSpec((5, blk), lambda i: (0, i)),
            pl.BlockSpec((g, 2), lambda i: (0, 0)),
        ],
        out_specs=pl.BlockSpec((1, 1, blk), lambda i: (i, 0, 0)),
        out_shape=jax.ShapeDtypeStruct((nblk, 1, blk), jnp.float32),
    )(lp, lt, grid)
    return out.reshape(-1)


_NSC = 3072


def kernel(loc_p, loc_t, grid):
    n = loc_p.shape[0]
    blk = 512
    ntot = ((n + blk - 1) // blk) * blk
    pad = ntot - n
    lp = jnp.pad(loc_p, ((0, pad), (0, 0)))
    lt = jnp.pad(loc_t, ((0, pad), (0, 0)))
    ntc = ntot - _NSC

    params = jnp.concatenate(
        _derived_params(lp[ntc:]) + _derived_params(lt[ntc:]), axis=0)
    sc_out = _sc_piou(_NSC, _NSC // _NW)(params, grid[:, 0], grid[:, 1])

    tc_out = _tc_piou(lp[:ntc].T, lt[:ntc].T, grid, blk)

    return jnp.concatenate([tc_out, sc_out])[:n]


# speedup vs baseline: 4.6680x; 1.0185x over previous
import functools

import jax
import jax.numpy as jnp
from jax import lax
from jax.experimental import pallas as pl
from jax.experimental.pallas import tpu as pltpu
from jax.experimental.pallas import tpu_sc as plsc

_K = 10.0
_EPS = 1e-9

_NC = 2
_NS = 16
_NW = _NC * _NS
_G = 1024


def _sc_piou(nsc, bpw):
    mesh = plsc.VectorSubcoreMesh(core_axis_name="c", subcore_axis_name="s")

    @functools.partial(
        pl.kernel,
        mesh=mesh,
        out_type=jax.ShapeDtypeStruct((nsc,), jnp.float32),
        scratch_types=[
            pltpu.VMEM((12 * bpw,), jnp.float32),
            pltpu.VMEM((_G,), jnp.float32),
            pltpu.VMEM((_G,), jnp.float32),
            pltpu.VMEM((bpw,), jnp.float32),
        ],
    )
    def body(params_hbm, gx_hbm, gy_hbm, out_hbm, pv, gxv, gyv, ov):
        wid = lax.axis_index("s") * _NC + lax.axis_index("c")
        base = wid * bpw
        pltpu.sync_copy(params_hbm.at[pl.ds(wid * 12 * bpw, 12 * bpw)], pv)
        pltpu.sync_copy(gx_hbm, gxv)
        pltpu.sync_copy(gy_hbm, gyv)

        def box_group(j, _):
            o = j * 16
            cxp = pv[pl.ds(0 * bpw + o, 16)]
            cyp = pv[pl.ds(1 * bpw + o, 16)]
            ctp = pv[pl.ds(2 * bpw + o, 16)]
            stp = pv[pl.ds(3 * bpw + o, 16)]
            wp = pv[pl.ds(4 * bpw + o, 16)]
            hp = pv[pl.ds(5 * bpw + o, 16)]
            cxt = pv[pl.ds(6 * bpw + o, 16)]
            cyt = pv[pl.ds(7 * bpw + o, 16)]
            ctt = pv[pl.ds(8 * bpw + o, 16)]
            stt = pv[pl.ds(9 * bpw + o, 16)]
            wt = pv[pl.ds(10 * bpw + o, 16)]
            ht = pv[pl.ds(11 * bpw + o, 16)]

            def denoms(gxs, gys):
                dxp = gxs - cxp
                dyp = gys - cyp
                ap = jnp.minimum(jnp.abs(dxp * ctp + dyp * stp) - wp, 20.0)
                bp = jnp.minimum(jnp.abs(dyp * ctp - dxp * stp) - hp, 20.0)
                dxt = gxs - cxt
                dyt = gys - cyt
                at = jnp.minimum(jnp.abs(dxt * ctt + dyt * stt) - wt, 20.0)
                bt = jnp.minimum(jnp.abs(dyt * ctt - dxt * stt) - ht, 20.0)
                dp = (1.0 + jnp.exp(ap)) * (1.0 + jnp.exp(bp))
                dt = (1.0 + jnp.exp(at)) * (1.0 + jnp.exp(bt))
                return dp, dt

            def gchunk(c, carry):
                accs = list(carry)
                gxc = gxv[pl.ds(c * 16, 16)]
                gyc = gyv[pl.ds(c * 16, 16)]
                for u in range(16):
                    dp, dt = denoms(gxc[u], gyc[u])
                    r = 1.0 / (dp * dt)
                    k = u % 4
                    accs[k] = accs[k] + r
                    accs[4 + k] = accs[4 + k] + (dp + dt - 1.0) * r
                return tuple(accs)

            z = jnp.zeros((16,), jnp.float32)
            accs = lax.fori_loop(0, _G // 16, gchunk, (z,) * 8)
            acc_i = (accs[0] + accs[1]) + (accs[2] + accs[3])
            acc_u = (accs[4] + accs[5]) + (accs[6] + accs[7])
            ov[pl.ds(o, 16)] = acc_i / (acc_u + _EPS)
            return 0

        lax.fori_loop(0, bpw // 16, box_group, 0)
        pltpu.sync_copy(ov, out_hbm.at[pl.ds(base, bpw)])

    return body


def _derived_params(loc):
    cx = loc[:, 0]
    cy = loc[:, 1]
    kw2 = (0.5 * _K) * loc[:, 2]
    kh2 = (0.5 * _K) * loc[:, 3]
    kct = _K * jnp.cos(loc[:, 4])
    kst = _K * jnp.sin(loc[:, 4])
    return [cx, cy, kct, kst, kw2, kh2]


def _tc_body(locp_ref, loct_ref, grid_ref, out_ref):
    gx = grid_ref[:, 0:1]
    gy = grid_ref[:, 1:2]
    kl2 = _K * 1.4426950408889634

    def denom(loc):
        cx = loc[0:1, :]
        cy = loc[1:2, :]
        th = loc[4:5, :]
        kct = kl2 * jnp.cos(th)
        kst = kl2 * jnp.sin(th)
        kw2 = (0.5 * kl2) * loc[2:3, :]
        kh2 = (0.5 * kl2) * loc[3:4, :]
        dx = gx - cx
        dy = gy - cy
        a = jnp.abs(dx * kct + dy * kst) - kw2
        b = jnp.abs(dy * kct - dx * kst) - kh2
        ea = jnp.exp2(jnp.minimum(a, 29.0))
        eb = jnp.exp2(jnp.minimum(b, 29.0))
        return (1.0 + ea) * (1.0 + eb)

    dp = denom(locp_ref[...])
    dt = denom(loct_ref[...])
    r = 1.0 / (dp * dt)
    inter = jnp.sum(r, axis=0)
    union = jnp.sum((dp + dt - 1.0) * r, axis=0)
    out_ref[...] = (inter / (union + _EPS))[None, None, :]


def _tc_piou(lp, lt, grid, blk):
    ntc = lp.shape[1]
    g = grid.shape[0]
    nblk = ntc // blk
    out = pl.pallas_call(
        _tc_body,
        grid=(nblk,),
        in_specs=[
            pl.BlockSpec((5, blk), lambda i: (0, i)),
            pl.BlockSpec((5, blk), lambda i: (0, i)),
            pl.BlockSpec((g, 2), lambda i: (0, 0)),
        ],
        out_specs=pl.BlockSpec((1, 1, blk), lambda i: (i, 0, 0)),
        out_shape=jax.ShapeDtypeStruct((nblk, 1, blk), jnp.float32),
    )(lp, lt, grid)
    return out.reshape(-1)


_NSC = 3072


def kernel(loc_p, loc_t, grid):
    n = loc_p.shape[0]
    blk = 512
    ntot = ((n + blk - 1) // blk) * blk
    pad = ntot - n
    lp = jnp.pad(loc_p, ((0, pad), (0, 0)))
    lt = jnp.pad(loc_t, ((0, pad), (0, 0)))
    ntc = ntot - _NSC

    bpw = _NSC // _NW
    p12 = jnp.stack(
        _derived_params(lp[ntc:]) + _derived_params(lt[ntc:]), axis=0)
    params = p12.reshape(12, _NW, bpw).transpose(1, 0, 2).reshape(-1)
    sc_out = _sc_piou(_NSC, bpw)(params, grid[:, 0], grid[:, 1])

    tc_out = _tc_piou(lp[:ntc].T, lt[:ntc].T, grid, blk)

    return jnp.concatenate([tc_out, sc_out])[:n]


# speedup vs baseline: 5.8651x; 1.2565x over previous
import functools

import jax
import jax.numpy as jnp
from jax import lax
from jax.experimental import pallas as pl
from jax.experimental.pallas import tpu as pltpu
from jax.experimental.pallas import tpu_sc as plsc

_K = 10.0
_EPS = 1e-9

_NC = 2
_NS = 16
_NW = _NC * _NS
_G = 1024


def _sc_piou(nsc, bpw):
    mesh = plsc.VectorSubcoreMesh(core_axis_name="c", subcore_axis_name="s")

    @functools.partial(
        pl.kernel,
        mesh=mesh,
        out_type=jax.ShapeDtypeStruct((nsc,), jnp.float32),
        scratch_types=[
            pltpu.VMEM((12 * bpw,), jnp.float32),
            pltpu.VMEM((64,), jnp.float32),
            pltpu.VMEM((8 * 32 * 16,), jnp.float32),
            pltpu.VMEM((bpw,), jnp.float32),
        ],
    )
    def body(params_hbm, xs_hbm, out_hbm, pv, xsv, tab, ov):
        wid = lax.axis_index("s") * _NC + lax.axis_index("c")
        base = wid * bpw
        pltpu.sync_copy(params_hbm.at[pl.ds(wid * 12 * bpw, 12 * bpw)], pv)
        pltpu.sync_copy(xs_hbm, xsv)

        xr0 = xsv[pl.ds(0, 16)]
        xr1 = xsv[pl.ds(16, 16)]
        xc0 = xsv[pl.ds(32, 16)]
        xc1 = xsv[pl.ds(48, 16)]
        xr = [xr0[u] for u in range(16)] + [xr1[u] for u in range(16)]
        xc = [xc0[u] for u in range(16)] + [xc1[u] for u in range(16)]

        def box_group(j, _):
            o = j * 16
            cxp = pv[pl.ds(0 * bpw + o, 16)]
            cyp = pv[pl.ds(1 * bpw + o, 16)]
            ctp = pv[pl.ds(2 * bpw + o, 16)]
            stp = pv[pl.ds(3 * bpw + o, 16)]
            wp = pv[pl.ds(4 * bpw + o, 16)]
            hp = pv[pl.ds(5 * bpw + o, 16)]
            cxt = pv[pl.ds(6 * bpw + o, 16)]
            cyt = pv[pl.ds(7 * bpw + o, 16)]
            ctt = pv[pl.ds(8 * bpw + o, 16)]
            stt = pv[pl.ds(9 * bpw + o, 16)]
            wt = pv[pl.ds(10 * bpw + o, 16)]
            ht = pv[pl.ds(11 * bpw + o, 16)]

            for i in range(32):
                dxp = xr[i] - cxp
                dxt = xr[i] - cxt
                tab[pl.ds(0 * 512 + i * 16, 16)] = dxp * ctp
                tab[pl.ds(1 * 512 + i * 16, 16)] = dxp * stp
                tab[pl.ds(2 * 512 + i * 16, 16)] = dxt * ctt
                tab[pl.ds(3 * 512 + i * 16, 16)] = dxt * stt
            for i in range(32):
                dyp = xc[i] - cyp
                dyt = xc[i] - cyt
                tab[pl.ds(4 * 512 + i * 16, 16)] = dyp * stp
                tab[pl.ds(5 * 512 + i * 16, 16)] = dyp * ctp
                tab[pl.ds(6 * 512 + i * 16, 16)] = dyt * stt
                tab[pl.ds(7 * 512 + i * 16, 16)] = dyt * ctt

            def row(i, carry):
                xcp = tab[pl.ds(0 * 512 + i * 16, 16)]
                xsp = tab[pl.ds(1 * 512 + i * 16, 16)]
                xct = tab[pl.ds(2 * 512 + i * 16, 16)]
                xst = tab[pl.ds(3 * 512 + i * 16, 16)]

                def col4(jc, carry2):
                    accs = list(carry2)
                    for v in range(4):
                        off = jc * 64 + v * 16
                        ysp = tab[pl.ds(4 * 512 + off, 16)]
                        ycp = tab[pl.ds(5 * 512 + off, 16)]
                        yst = tab[pl.ds(6 * 512 + off, 16)]
                        yct = tab[pl.ds(7 * 512 + off, 16)]
                        ap = jnp.minimum(jnp.abs(xcp + ysp) - wp, 20.0)
                        bp = jnp.minimum(jnp.abs(ycp - xsp) - hp, 20.0)
                        at = jnp.minimum(jnp.abs(xct + yst) - wt, 20.0)
                        bt = jnp.minimum(jnp.abs(yct - xst) - ht, 20.0)
                        dp = (1.0 + jnp.exp(ap)) * (1.0 + jnp.exp(bp))
                        dt = (1.0 + jnp.exp(at)) * (1.0 + jnp.exp(bt))
                        r = 1.0 / (dp * dt)
                        accs[v] = accs[v] + r
                        accs[4 + v] = accs[4 + v] + (dp + dt - 1.0) * r
                    return tuple(accs)

                return lax.fori_loop(0, 8, col4, carry)

            z = jnp.zeros((16,), jnp.float32)
            accs = lax.fori_loop(0, 32, row, (z,) * 8)
            acc_i = (accs[0] + accs[1]) + (accs[2] + accs[3])
            acc_u = (accs[4] + accs[5]) + (accs[6] + accs[7])
            ov[pl.ds(o, 16)] = acc_i / (acc_u + _EPS)
            return 0

        lax.fori_loop(0, bpw // 16, box_group, 0)
        pltpu.sync_copy(ov, out_hbm.at[pl.ds(base, bpw)])

    return body


def _derived_params(loc):
    cx = loc[:, 0]
    cy = loc[:, 1]
    kw2 = (0.5 * _K) * loc[:, 2]
    kh2 = (0.5 * _K) * loc[:, 3]
    kct = _K * jnp.cos(loc[:, 4])
    kst = _K * jnp.sin(loc[:, 4])
    return [cx, cy, kct, kst, kw2, kh2]


def _tc_body(locp_ref, loct_ref, grid_ref, out_ref):
    gx = grid_ref[:, 0:1]
    gy = grid_ref[:, 1:2]
    kl2 = _K * 1.4426950408889634

    def denom(loc):
        cx = loc[0:1, :]
        cy = loc[1:2, :]
        th = loc[4:5, :]
        kct = kl2 * jnp.cos(th)
        kst = kl2 * jnp.sin(th)
        kw2 = (0.5 * kl2) * loc[2:3, :]
        kh2 = (0.5 * kl2) * loc[3:4, :]
        dx = gx - cx
        dy = gy - cy
        a = jnp.abs(dx * kct + dy * kst) - kw2
        b = jnp.abs(dy * kct - dx * kst) - kh2
        ea = jnp.exp2(jnp.minimum(a, 29.0))
        eb = jnp.exp2(jnp.minimum(b, 29.0))
        return (1.0 + ea) * (1.0 + eb)

    dp = denom(locp_ref[...])
    dt = denom(loct_ref[...])
    r = 1.0 / (dp * dt)
    inter = jnp.sum(r, axis=0)
    union = jnp.sum((dp + dt - 1.0) * r, axis=0)
    out_ref[...] = (inter / (union + _EPS))[None, None, :]


def _tc_piou(lp, lt, grid, blk):
    ntc = lp.shape[1]
    g = grid.shape[0]
    nblk = ntc // blk
    out = pl.pallas_call(
        _tc_body,
        grid=(nblk,),
        in_specs=[
            pl.BlockSpec((5, blk), lambda i: (0, i)),
            pl.BlockSpec((5, blk), lambda i: (0, i)),
            pl.BlockSpec((g, 2), lambda i: (0, 0)),
        ],
        out_specs=pl.BlockSpec((1, 1, blk), lambda i: (i, 0, 0)),
        out_shape=jax.ShapeDtypeStruct((nblk, 1, blk), jnp.float32),
    )(lp, lt, grid)
    return out.reshape(-1)


_NSC = 3072


def kernel(loc_p, loc_t, grid):
    n = loc_p.shape[0]
    blk = 512
    ntot = ((n + blk - 1) // blk) * blk
    pad = ntot - n
    lp = jnp.pad(loc_p, ((0, pad), (0, 0)))
    lt = jnp.pad(loc_t, ((0, pad), (0, 0)))
    ntc = ntot - _NSC

    bpw = _NSC // _NW
    p12 = jnp.stack(
        _derived_params(lp[ntc:]) + _derived_params(lt[ntc:]), axis=0)
    params = p12.reshape(12, _NW, bpw).transpose(1, 0, 2).reshape(-1)
    xs = jnp.concatenate([grid[::32, 0], grid[:32, 1]])
    sc_out = _sc_piou(_NSC, bpw)(params, xs)

    tc_out = _tc_piou(lp[:ntc].T, lt[:ntc].T, grid, blk)

    return jnp.concatenate([tc_out, sc_out])[:n]


# speedup vs baseline: 6.4147x; 1.0937x over previous
import functools

import jax
import jax.numpy as jnp
from jax import lax
from jax.experimental import pallas as pl
from jax.experimental.pallas import tpu as pltpu
from jax.experimental.pallas import tpu_sc as plsc

_K = 10.0
_EPS = 1e-9

_NC = 2
_NS = 16
_NW = _NC * _NS
_G = 1024


def _sc_piou(nsc, bpw):
    mesh = plsc.VectorSubcoreMesh(core_axis_name="c", subcore_axis_name="s")

    @functools.partial(
        pl.kernel,
        mesh=mesh,
        out_type=jax.ShapeDtypeStruct((nsc,), jnp.float32),
        scratch_types=[
            pltpu.VMEM((12 * bpw,), jnp.float32),
            pltpu.VMEM((64,), jnp.float32),
            pltpu.VMEM((8 * 32 * 16,), jnp.float32),
            pltpu.VMEM((bpw,), jnp.float32),
        ],
    )
    def body(params_hbm, xs_hbm, out_hbm, pv, xsv, tab, ov):
        wid = lax.axis_index("s") * _NC + lax.axis_index("c")
        base = wid * bpw
        pltpu.sync_copy(params_hbm.at[pl.ds(wid * 12 * bpw, 12 * bpw)], pv)
        pltpu.sync_copy(xs_hbm, xsv)

        xr0 = xsv[pl.ds(0, 16)]
        xr1 = xsv[pl.ds(16, 16)]
        xc0 = xsv[pl.ds(32, 16)]
        xc1 = xsv[pl.ds(48, 16)]
        xr = [xr0[u] for u in range(16)] + [xr1[u] for u in range(16)]
        xc = [xc0[u] for u in range(16)] + [xc1[u] for u in range(16)]

        def box_group(j, _):
            o = j * 16
            cxp = pv[pl.ds(0 * bpw + o, 16)]
            cyp = pv[pl.ds(1 * bpw + o, 16)]
            ctp = pv[pl.ds(2 * bpw + o, 16)]
            stp = pv[pl.ds(3 * bpw + o, 16)]
            wp = pv[pl.ds(4 * bpw + o, 16)]
            hp = pv[pl.ds(5 * bpw + o, 16)]
            cxt = pv[pl.ds(6 * bpw + o, 16)]
            cyt = pv[pl.ds(7 * bpw + o, 16)]
            ctt = pv[pl.ds(8 * bpw + o, 16)]
            stt = pv[pl.ds(9 * bpw + o, 16)]
            wt = pv[pl.ds(10 * bpw + o, 16)]
            ht = pv[pl.ds(11 * bpw + o, 16)]

            for i in range(32):
                dxp = xr[i] - cxp
                dxt = xr[i] - cxt
                tab[pl.ds(0 * 512 + i * 16, 16)] = dxp * ctp
                tab[pl.ds(1 * 512 + i * 16, 16)] = dxp * stp
                tab[pl.ds(2 * 512 + i * 16, 16)] = dxt * ctt
                tab[pl.ds(3 * 512 + i * 16, 16)] = dxt * stt
            for i in range(32):
                dyp = xc[i] - cyp
                dyt = xc[i] - cyt
                tab[pl.ds(4 * 512 + i * 16, 16)] = dyp * stp
                tab[pl.ds(5 * 512 + i * 16, 16)] = dyp * ctp
                tab[pl.ds(6 * 512 + i * 16, 16)] = dyt * stt
                tab[pl.ds(7 * 512 + i * 16, 16)] = dyt * ctt

            def row(i, carry):
                xcp = tab[pl.ds(0 * 512 + i * 16, 16)]
                xsp = tab[pl.ds(1 * 512 + i * 16, 16)]
                xct = tab[pl.ds(2 * 512 + i * 16, 16)]
                xst = tab[pl.ds(3 * 512 + i * 16, 16)]

                def col4(jc, carry2):
                    accs = list(carry2)
                    for v in range(4):
                        off = jc * 64 + v * 16
                        ysp = tab[pl.ds(4 * 512 + off, 16)]
                        ycp = tab[pl.ds(5 * 512 + off, 16)]
                        yst = tab[pl.ds(6 * 512 + off, 16)]
                        yct = tab[pl.ds(7 * 512 + off, 16)]
                        ap = jnp.minimum(jnp.abs(xcp + ysp) - wp, 20.0)
                        bp = jnp.minimum(jnp.abs(ycp - xsp) - hp, 20.0)
                        at = jnp.minimum(jnp.abs(xct + yst) - wt, 20.0)
                        bt = jnp.minimum(jnp.abs(yct - xst) - ht, 20.0)
                        dp = (1.0 + jnp.exp(ap)) * (1.0 + jnp.exp(bp))
                        dt = (1.0 + jnp.exp(at)) * (1.0 + jnp.exp(bt))
                        r = 1.0 / (dp * dt)
                        accs[v] = accs[v] + r
                        accs[4 + v] = accs[4 + v] + (dp + dt - 1.0) * r
                    return tuple(accs)

                return lax.fori_loop(0, 8, col4, carry)

            z = jnp.zeros((16,), jnp.float32)
            accs = lax.fori_loop(0, 32, row, (z,) * 8)
            acc_i = (accs[0] + accs[1]) + (accs[2] + accs[3])
            acc_u = (accs[4] + accs[5]) + (accs[6] + accs[7])
            ov[pl.ds(o, 16)] = acc_i / (acc_u + _EPS)
            return 0

        lax.fori_loop(0, bpw // 16, box_group, 0)
        pltpu.sync_copy(ov, out_hbm.at[pl.ds(base, bpw)])

    return body


def _derived_params(loc):
    cx = loc[:, 0]
    cy = loc[:, 1]
    kw2 = (0.5 * _K) * loc[:, 2]
    kh2 = (0.5 * _K) * loc[:, 3]
    kct = _K * jnp.cos(loc[:, 4])
    kst = _K * jnp.sin(loc[:, 4])
    return [cx, cy, kct, kst, kw2, kh2]


def _tc_body(locp_ref, loct_ref, grid_ref, out_ref):
    gx = grid_ref[:, 0:1]
    gy = grid_ref[:, 1:2]
    kl2 = _K * 1.4426950408889634

    def denom(loc):
        cx = loc[0:1, :]
        cy = loc[1:2, :]
        th = loc[4:5, :]
        kct = kl2 * jnp.cos(th)
        kst = kl2 * jnp.sin(th)
        kw2 = (0.5 * kl2) * loc[2:3, :]
        kh2 = (0.5 * kl2) * loc[3:4, :]
        dx = gx - cx
        dy = gy - cy
        a = jnp.abs(dx * kct + dy * kst) - kw2
        b = jnp.abs(dy * kct - dx * kst) - kh2
        ea = jnp.exp2(jnp.minimum(a, 29.0))
        eb = jnp.exp2(jnp.minimum(b, 29.0))
        return (1.0 + ea) * (1.0 + eb)

    dp = denom(locp_ref[...])
    dt = denom(loct_ref[...])
    r = 1.0 / (dp * dt)
    inter = jnp.sum(r, axis=0)
    union = jnp.sum((dp + dt - 1.0) * r, axis=0)
    out_ref[...] = (inter / (union + _EPS))[None, None, :]


def _tc_piou(lp, lt, grid, blk):
    ntc = lp.shape[1]
    g = grid.shape[0]
    nblk = ntc // blk
    out = pl.pallas_call(
        _tc_body,
        grid=(nblk,),
        in_specs=[
            pl.BlockSpec((5, blk), lambda i: (0, i)),
            pl.BlockSpec((5, blk), lambda i: (0, i)),
            pl.BlockSpec((g, 2), lambda i: (0, 0)),
        ],
        out_specs=pl.BlockSpec((1, 1, blk), lambda i: (i, 0, 0)),
        out_shape=jax.ShapeDtypeStruct((nblk, 1, blk), jnp.float32),
    )(lp, lt, grid)
    return out.reshape(-1)


_NSC = 6144


def kernel(loc_p, loc_t, grid):
    n = loc_p.shape[0]
    blk = 512
    ntot = ((n + blk - 1) // blk) * blk
    pad = ntot - n
    lp = jnp.pad(loc_p, ((0, pad), (0, 0)))
    lt = jnp.pad(loc_t, ((0, pad), (0, 0)))
    ntc = ntot - _NSC

    bpw = _NSC // _NW
    p12 = jnp.stack(
        _derived_params(lp[ntc:]) + _derived_params(lt[ntc:]), axis=0)
    params = p12.reshape(12, _NW, bpw).transpose(1, 0, 2).reshape(-1)
    xs = jnp.concatenate([grid[::32, 0], grid[:32, 1]])
    sc_out = _sc_piou(_NSC, bpw)(params, xs)

    tc_out = _tc_piou(lp[:ntc].T, lt[:ntc].T, grid, blk)

    return jnp.concatenate([tc_out, sc_out])[:n]


# speedup vs baseline: 6.4455x; 1.0048x over previous
import functools

import jax
import jax.numpy as jnp
from jax import lax
from jax.experimental import pallas as pl
from jax.experimental.pallas import tpu as pltpu
from jax.experimental.pallas import tpu_sc as plsc

_K = 10.0
_EPS = 1e-9

_NC = 2
_NS = 16
_NW = _NC * _NS
_G = 1024


def _sc_piou(nsc, bpw):
    mesh = plsc.VectorSubcoreMesh(core_axis_name="c", subcore_axis_name="s")

    @functools.partial(
        pl.kernel,
        mesh=mesh,
        out_type=jax.ShapeDtypeStruct((nsc,), jnp.float32),
        scratch_types=[
            pltpu.VMEM((12 * bpw,), jnp.float32),
            pltpu.VMEM((64,), jnp.float32),
            pltpu.VMEM((8 * 32 * 16,), jnp.float32),
            pltpu.VMEM((bpw,), jnp.float32),
        ],
    )
    def body(params_hbm, xs_hbm, out_hbm, pv, xsv, tab, ov):
        wid = lax.axis_index("s") * _NC + lax.axis_index("c")
        base = wid * bpw
        pltpu.sync_copy(params_hbm.at[pl.ds(wid * 12 * bpw, 12 * bpw)], pv)
        pltpu.sync_copy(xs_hbm, xsv)

        xr0 = xsv[pl.ds(0, 16)]
        xr1 = xsv[pl.ds(16, 16)]
        xc0 = xsv[pl.ds(32, 16)]
        xc1 = xsv[pl.ds(48, 16)]
        xr = [xr0[u] for u in range(16)] + [xr1[u] for u in range(16)]
        xc = [xc0[u] for u in range(16)] + [xc1[u] for u in range(16)]

        def box_group(j, _):
            o = j * 16
            cxp = pv[pl.ds(0 * bpw + o, 16)]
            cyp = pv[pl.ds(1 * bpw + o, 16)]
            ctp = pv[pl.ds(2 * bpw + o, 16)]
            stp = pv[pl.ds(3 * bpw + o, 16)]
            wp = pv[pl.ds(4 * bpw + o, 16)]
            hp = pv[pl.ds(5 * bpw + o, 16)]
            cxt = pv[pl.ds(6 * bpw + o, 16)]
            cyt = pv[pl.ds(7 * bpw + o, 16)]
            ctt = pv[pl.ds(8 * bpw + o, 16)]
            stt = pv[pl.ds(9 * bpw + o, 16)]
            wt = pv[pl.ds(10 * bpw + o, 16)]
            ht = pv[pl.ds(11 * bpw + o, 16)]

            for i in range(32):
                dxp = xr[i] - cxp
                dxt = xr[i] - cxt
                tab[pl.ds(0 * 512 + i * 16, 16)] = dxp * ctp
                tab[pl.ds(1 * 512 + i * 16, 16)] = dxp * stp
                tab[pl.ds(2 * 512 + i * 16, 16)] = dxt * ctt
                tab[pl.ds(3 * 512 + i * 16, 16)] = dxt * stt
            for i in range(32):
                dyp = xc[i] - cyp
                dyt = xc[i] - cyt
                tab[pl.ds(4 * 512 + i * 16, 16)] = dyp * stp
                tab[pl.ds(5 * 512 + i * 16, 16)] = dyp * ctp
                tab[pl.ds(6 * 512 + i * 16, 16)] = dyt * stt
                tab[pl.ds(7 * 512 + i * 16, 16)] = dyt * ctt

            def row(i, carry):
                xcp = tab[pl.ds(0 * 512 + i * 16, 16)]
                xsp = tab[pl.ds(1 * 512 + i * 16, 16)]
                xct = tab[pl.ds(2 * 512 + i * 16, 16)]
                xst = tab[pl.ds(3 * 512 + i * 16, 16)]

                def col4(jc, carry2):
                    accs = list(carry2)
                    for v in range(4):
                        off = jc * 64 + v * 16
                        ysp = tab[pl.ds(4 * 512 + off, 16)]
                        ycp = tab[pl.ds(5 * 512 + off, 16)]
                        yst = tab[pl.ds(6 * 512 + off, 16)]
                        yct = tab[pl.ds(7 * 512 + off, 16)]
                        ap = jnp.minimum(jnp.abs(xcp + ysp) - wp, 20.0)
                        bp = jnp.minimum(jnp.abs(ycp - xsp) - hp, 20.0)
                        at = jnp.minimum(jnp.abs(xct + yst) - wt, 20.0)
                        bt = jnp.minimum(jnp.abs(yct - xst) - ht, 20.0)
                        dp = (1.0 + jnp.exp(ap)) * (1.0 + jnp.exp(bp))
                        dt = (1.0 + jnp.exp(at)) * (1.0 + jnp.exp(bt))
                        r = 1.0 / (dp * dt)
                        accs[v] = accs[v] + r
                        accs[4 + v] = accs[4 + v] + (dp + dt - 1.0) * r
                    return tuple(accs)

                return lax.fori_loop(0, 8, col4, carry)

            z = jnp.zeros((16,), jnp.float32)
            accs = lax.fori_loop(0, 32, row, (z,) * 8)
            acc_i = (accs[0] + accs[1]) + (accs[2] + accs[3])
            acc_u = (accs[4] + accs[5]) + (accs[6] + accs[7])
            ov[pl.ds(o, 16)] = acc_i / (acc_u + _EPS)
            return 0

        lax.fori_loop(0, bpw // 16, box_group, 0)
        pltpu.sync_copy(ov, out_hbm.at[pl.ds(base, bpw)])

    return body


def _derived_params(loc):
    cx = loc[:, 0]
    cy = loc[:, 1]
    kw2 = (0.5 * _K) * loc[:, 2]
    kh2 = (0.5 * _K) * loc[:, 3]
    kct = _K * jnp.cos(loc[:, 4])
    kst = _K * jnp.sin(loc[:, 4])
    return [cx, cy, kct, kst, kw2, kh2]


def _tc_body(locp_ref, loct_ref, grid_ref, out_ref):
    gx = grid_ref[:, 0:1]
    gy = grid_ref[:, 1:2]
    kl2 = _K * 1.4426950408889634

    def denom(loc):
        cx = loc[0:1, :]
        cy = loc[1:2, :]
        th = loc[4:5, :]
        kct = kl2 * jnp.cos(th)
        kst = kl2 * jnp.sin(th)
        kw2 = (0.5 * kl2) * loc[2:3, :]
        kh2 = (0.5 * kl2) * loc[3:4, :]
        dx = gx - cx
        dy = gy - cy
        a = jnp.abs(dx * kct + dy * kst) - kw2
        b = jnp.abs(dy * kct - dx * kst) - kh2
        ea = jnp.exp2(jnp.minimum(a, 29.0))
        eb = jnp.exp2(jnp.minimum(b, 29.0))
        return (1.0 + ea) * (1.0 + eb)

    dp = denom(locp_ref[...])
    dt = denom(loct_ref[...])
    r = 1.0 / (dp * dt)
    inter = jnp.sum(r, axis=0)
    union = jnp.sum((dp + dt - 1.0) * r, axis=0)
    out_ref[...] = (inter / (union + _EPS))[None, None, :]


def _tc_piou(lp, lt, grid, blk):
    ntc = lp.shape[1]
    g = grid.shape[0]
    nblk = ntc // blk
    out = pl.pallas_call(
        _tc_body,
        grid=(nblk,),
        in_specs=[
            pl.BlockSpec((5, blk), lambda i: (0, i)),
            pl.BlockSpec((5, blk), lambda i: (0, i)),
            pl.BlockSpec((g, 2), lambda i: (0, 0)),
        ],
        out_specs=pl.BlockSpec((1, 1, blk), lambda i: (i, 0, 0)),
        out_shape=jax.ShapeDtypeStruct((nblk, 1, blk), jnp.float32),
    )(lp, lt, grid)
    return out.reshape(-1)


_NSC = 5632


def kernel(loc_p, loc_t, grid):
    n = loc_p.shape[0]
    blk = 512
    ntot = ((n + blk - 1) // blk) * blk
    pad = ntot - n
    lp = jnp.pad(loc_p, ((0, pad), (0, 0)))
    lt = jnp.pad(loc_t, ((0, pad), (0, 0)))
    ntc = ntot - _NSC

    bpw = _NSC // _NW
    p12 = jnp.stack(
        _derived_params(lp[ntc:]) + _derived_params(lt[ntc:]), axis=0)
    params = p12.reshape(12, _NW, bpw).transpose(1, 0, 2).reshape(-1)
    xs = jnp.concatenate([grid[::32, 0], grid[:32, 1]])
    sc_out = _sc_piou(_NSC, bpw)(params, xs)

    tc_out = _tc_piou(lp[:ntc].T, lt[:ntc].T, grid, blk)

    return jnp.concatenate([tc_out, sc_out])[:n]


# speedup vs baseline: 7.4466x; 1.1553x over previous
import functools

import jax
import jax.numpy as jnp
from jax import lax
from jax.experimental import pallas as pl
from jax.experimental.pallas import tpu as pltpu
from jax.experimental.pallas import tpu_sc as plsc

_K = 10.0
_EPS = 1e-9

_NC = 2
_NS = 16
_NW = _NC * _NS
_G = 1024


def _sc_piou(nsc, bpw):
    mesh = plsc.VectorSubcoreMesh(core_axis_name="c", subcore_axis_name="s")

    @functools.partial(
        pl.kernel,
        mesh=mesh,
        out_type=jax.ShapeDtypeStruct((nsc,), jnp.float32),
        scratch_types=[
            pltpu.VMEM((12 * bpw,), jnp.float32),
            pltpu.VMEM((64,), jnp.float32),
            pltpu.VMEM((8 * 32 * 16,), jnp.float32),
            pltpu.VMEM((bpw,), jnp.float32),
        ],
    )
    def body(params_hbm, xs_hbm, out_hbm, pv, xsv, tab, ov):
        wid = lax.axis_index("s") * _NC + lax.axis_index("c")
        base = wid * bpw
        pltpu.sync_copy(params_hbm.at[pl.ds(wid * 12 * bpw, 12 * bpw)], pv)
        pltpu.sync_copy(xs_hbm, xsv)

        xr0 = xsv[pl.ds(0, 16)]
        xr1 = xsv[pl.ds(16, 16)]
        xc0 = xsv[pl.ds(32, 16)]
        xc1 = xsv[pl.ds(48, 16)]
        xr = [xr0[u] for u in range(16)] + [xr1[u] for u in range(16)]
        xc = [xc0[u] for u in range(16)] + [xc1[u] for u in range(16)]

        def box_group(j, _):
            o = j * 16
            cxp = pv[pl.ds(0 * bpw + o, 16)]
            cyp = pv[pl.ds(1 * bpw + o, 16)]
            ctp = pv[pl.ds(2 * bpw + o, 16)]
            stp = pv[pl.ds(3 * bpw + o, 16)]
            wp = pv[pl.ds(4 * bpw + o, 16)]
            hp = pv[pl.ds(5 * bpw + o, 16)]
            cxt = pv[pl.ds(6 * bpw + o, 16)]
            cyt = pv[pl.ds(7 * bpw + o, 16)]
            ctt = pv[pl.ds(8 * bpw + o, 16)]
            stt = pv[pl.ds(9 * bpw + o, 16)]
            wt = pv[pl.ds(10 * bpw + o, 16)]
            ht = pv[pl.ds(11 * bpw + o, 16)]

            for i in range(32):
                dxp = xr[i] - cxp
                dxt = xr[i] - cxt
                tab[pl.ds(0 * 512 + i * 16, 16)] = dxp * ctp
                tab[pl.ds(1 * 512 + i * 16, 16)] = dxp * stp
                tab[pl.ds(2 * 512 + i * 16, 16)] = dxt * ctt
                tab[pl.ds(3 * 512 + i * 16, 16)] = dxt * stt
            for i in range(32):
                dyp = xc[i] - cyp
                dyt = xc[i] - cyt
                tab[pl.ds(4 * 512 + i * 16, 16)] = dyp * stp
                tab[pl.ds(5 * 512 + i * 16, 16)] = dyp * ctp
                tab[pl.ds(6 * 512 + i * 16, 16)] = dyt * stt
                tab[pl.ds(7 * 512 + i * 16, 16)] = dyt * ctt

            def row(i, carry):
                xcp = tab[pl.ds(0 * 512 + i * 16, 16)]
                xsp = tab[pl.ds(1 * 512 + i * 16, 16)]
                xct = tab[pl.ds(2 * 512 + i * 16, 16)]
                xst = tab[pl.ds(3 * 512 + i * 16, 16)]

                def col4(jc, carry2):
                    accs = list(carry2)
                    for v in range(4):
                        off = jc * 64 + v * 16
                        ysp = tab[pl.ds(4 * 512 + off, 16)]
                        ycp = tab[pl.ds(5 * 512 + off, 16)]
                        yst = tab[pl.ds(6 * 512 + off, 16)]
                        yct = tab[pl.ds(7 * 512 + off, 16)]
                        ap = jnp.minimum(jnp.abs(xcp + ysp) - wp, 20.0)
                        bp = jnp.minimum(jnp.abs(ycp - xsp) - hp, 20.0)
                        at = jnp.minimum(jnp.abs(xct + yst) - wt, 20.0)
                        bt = jnp.minimum(jnp.abs(yct - xst) - ht, 20.0)
                        dp = (1.0 + jnp.exp(ap)) * (1.0 + jnp.exp(bp))
                        dt = (1.0 + jnp.exp(at)) * (1.0 + jnp.exp(bt))
                        r = 1.0 / (dp * dt)
                        accs[v] = accs[v] + r
                        accs[4 + v] = accs[4 + v] + (dp + dt - 1.0) * r
                    return tuple(accs)

                return lax.fori_loop(0, 8, col4, carry)

            z = jnp.zeros((16,), jnp.float32)
            accs = lax.fori_loop(0, 32, row, (z,) * 8)
            acc_i = (accs[0] + accs[1]) + (accs[2] + accs[3])
            acc_u = (accs[4] + accs[5]) + (accs[6] + accs[7])
            ov[pl.ds(o, 16)] = acc_i / (acc_u + _EPS)
            return 0

        lax.fori_loop(0, bpw // 16, box_group, 0)
        pltpu.sync_copy(ov, out_hbm.at[pl.ds(base, bpw)])

    return body


def _derived_params(loc):
    cx = loc[:, 0]
    cy = loc[:, 1]
    kw2 = (0.5 * _K) * loc[:, 2]
    kh2 = (0.5 * _K) * loc[:, 3]
    kct = _K * jnp.cos(loc[:, 4])
    kst = _K * jnp.sin(loc[:, 4])
    return [cx, cy, kct, kst, kw2, kh2]


def _tc_body(locp_ref, loct_ref, xr_ref, xc_ref, out_ref):
    xr = xr_ref[...]
    xc = xc_ref[...]
    kl2 = _K * 1.4426950408889634

    def tables(loc):
        cx = loc[0:1, :]
        cy = loc[1:2, :]
        th = loc[4:5, :]
        kct = kl2 * jnp.cos(th)
        kst = kl2 * jnp.sin(th)
        kw2 = (0.5 * kl2) * loc[2:3, :]
        kh2 = (0.5 * kl2) * loc[3:4, :]
        dx = xr - cx
        dy = xc - cy
        return dx * kct, dx * kst, dy * kst, dy * kct, kw2, kh2

    xct_p, xs_p, ys_p, yct_p, kw2p, kh2p = tables(locp_ref[...])
    xct_t, xs_t, ys_t, yct_t, kw2t, kh2t = tables(loct_ref[...])

    acc_i = jnp.zeros_like(ys_p)
    acc_u = jnp.zeros_like(ys_p)
    for i in range(32):
        ap = jnp.abs(xct_p[i:i + 1, :] + ys_p) - kw2p
        bp = jnp.abs(yct_p - xs_p[i:i + 1, :]) - kh2p
        at = jnp.abs(xct_t[i:i + 1, :] + ys_t) - kw2t
        bt = jnp.abs(yct_t - xs_t[i:i + 1, :]) - kh2t
        dp = ((1.0 + jnp.exp2(jnp.minimum(ap, 29.0)))
              * (1.0 + jnp.exp2(jnp.minimum(bp, 29.0))))
        dt = ((1.0 + jnp.exp2(jnp.minimum(at, 29.0)))
              * (1.0 + jnp.exp2(jnp.minimum(bt, 29.0))))
        r = 1.0 / (dp * dt)
        acc_i = acc_i + r
        acc_u = acc_u + (dp + dt - 1.0) * r

    inter = jnp.sum(acc_i, axis=0)
    union = jnp.sum(acc_u, axis=0)
    out_ref[...] = (inter / (union + _EPS))[None, None, :]


def _tc_piou(lp, lt, xr, xc, blk):
    ntc = lp.shape[1]
    nblk = ntc // blk
    out = pl.pallas_call(
        _tc_body,
        grid=(nblk,),
        in_specs=[
            pl.BlockSpec((5, blk), lambda i: (0, i)),
            pl.BlockSpec((5, blk), lambda i: (0, i)),
            pl.BlockSpec((32, 1), lambda i: (0, 0)),
            pl.BlockSpec((32, 1), lambda i: (0, 0)),
        ],
        out_specs=pl.BlockSpec((1, 1, blk), lambda i: (i, 0, 0)),
        out_shape=jax.ShapeDtypeStruct((nblk, 1, blk), jnp.float32),
    )(lp, lt, xr, xc)
    return out.reshape(-1)


_NSC = 4608


def kernel(loc_p, loc_t, grid):
    n = loc_p.shape[0]
    blk = 512
    ntot = ((n + blk - 1) // blk) * blk
    pad = ntot - n
    lp = jnp.pad(loc_p, ((0, pad), (0, 0)))
    lt = jnp.pad(loc_t, ((0, pad), (0, 0)))
    ntc = ntot - _NSC

    bpw = _NSC // _NW
    p12 = jnp.stack(
        _derived_params(lp[ntc:]) + _derived_params(lt[ntc:]), axis=0)
    params = p12.reshape(12, _NW, bpw).transpose(1, 0, 2).reshape(-1)
    xs = jnp.concatenate([grid[::32, 0], grid[:32, 1]])
    sc_out = _sc_piou(_NSC, bpw)(params, xs)

    xr = grid[::32, 0:1]
    xc = grid[:32, 1:2]
    tc_out = _tc_piou(lp[:ntc].T, lt[:ntc].T, xr, xc, blk)

    return jnp.concatenate([tc_out, sc_out])[:n]
